# Initial kernel scaffold; baseline (speedup 1.0000x reference)
import jax
import jax.numpy as jnp
from jax.experimental import pallas as pl


def kernel(x, edge_index, W1, b1, W2, b2):
    raise NotImplementedError("write your pallas kernel here")


# speedup vs baseline: 19.8535x; 19.8535x over previous
import functools

import jax
import jax.numpy as jnp
from jax import lax
from jax.experimental import pallas as pl
from jax.experimental.pallas import tpu as pltpu
from jax.experimental.pallas import tpu_sc as plsc

NC = 2
NS = 16
L = 16
TILES = NC * NS
EB = 128
RB = 512


def _sc_mesh():
    return plsc.VectorSubcoreMesh(core_axis_name="c", subcore_axis_name="s")


_SC_PARAMS = pltpu.CompilerParams(use_tc_tiling_on_sc=False)


@functools.lru_cache(maxsize=None)
def _make_deg(npad, nb):
    rpt = npad // NS

    @functools.partial(
        pl.kernel,
        out_type=jax.ShapeDtypeStruct((NC, npad), jnp.float32),
        mesh=_sc_mesh(),
        compiler_params=_SC_PARAMS,
        scratch_types=[
            pltpu.VMEM((nb, EB), jnp.int32),
            pltpu.VMEM((EB,), jnp.float32),
            pltpu.VMEM((rpt,), jnp.float32),
            pltpu.VMEM_SHARED((npad,), jnp.float32),
        ],
    )
    def deg_kernel(dst_hbm, out_hbm, dst_v, ones_v, zer_v, acc):
        c = lax.axis_index("c")
        s = lax.axis_index("s")
        w = c * NS + s

        @pl.loop(0, EB // L)
        def _(i):
            ones_v[pl.ds(i * L, L)] = jnp.ones((L,), jnp.float32)

        @pl.loop(0, rpt // L)
        def _(i):
            zer_v[pl.ds(i * L, L)] = jnp.zeros((L,), jnp.float32)

        pltpu.sync_copy(zer_v, acc.at[pl.ds(s * rpt, rpt)])
        plsc.subcore_barrier()

        pltpu.sync_copy(dst_hbm.at[w], dst_v)

        @pl.loop(0, nb)
        def _(j):
            pltpu.sync_copy(ones_v, acc.at[dst_v.at[j]], add=True)

        plsc.subcore_barrier()
        pltpu.sync_copy(acc.at[pl.ds(s * rpt, rpt)],
                        out_hbm.at[c, pl.ds(s * rpt, rpt)])

    return deg_kernel


@functools.lru_cache(maxsize=None)
def _make_prop(npad, nb, wd):
    rpt = npad // NS

    @functools.partial(
        pl.kernel,
        out_type=jax.ShapeDtypeStruct((NC, npad, wd), jnp.float32),
        mesh=_sc_mesh(),
        compiler_params=_SC_PARAMS,
        scratch_types=[
            pltpu.VMEM((nb, EB), jnp.int32),
            pltpu.VMEM((nb, EB), jnp.int32),
            pltpu.VMEM((EB, wd), jnp.float32),
            pltpu.VMEM_SHARED((npad, wd), jnp.float32),
            pltpu.SemaphoreType.DMA,
        ],
    )
    def prop_kernel(y_hbm, src_hbm, dst_hbm, out_hbm,
                    src_v, dst_v, rows_v, acc, sem):
        c = lax.axis_index("c")
        s = lax.axis_index("s")
        w = c * NS + s

        @pl.loop(0, EB)
        def _(r):
            @pl.loop(0, wd // L)
            def _(q):
                rows_v[r, pl.ds(q * L, L)] = jnp.zeros((L,), jnp.float32)

        @pl.loop(0, rpt // EB)
        def _(i):
            pltpu.sync_copy(rows_v, acc.at[pl.ds(s * rpt + i * EB, EB)])

        plsc.subcore_barrier()

        pltpu.sync_copy(src_hbm.at[w], src_v)
        pltpu.sync_copy(dst_hbm.at[w], dst_v)

        @pl.loop(0, nb)
        def _(j):
            pltpu.async_copy(y_hbm.at[src_v.at[j]], rows_v, sem).wait()
            pltpu.sync_copy(rows_v, acc.at[dst_v.at[j]], add=True)

        plsc.subcore_barrier()
        pltpu.sync_copy(acc.at[pl.ds(s * rpt, rpt)],
                        out_hbm.at[c, pl.ds(s * rpt, rpt)])

    return prop_kernel


def _tc1(degp, xp, W1):
    npad = xp.shape[0]

    def body(deg_ref, x_ref, w1_ref, y_ref):
        dinv = lax.rsqrt(deg_ref[0] + deg_ref[1] + 1.0)
        y_ref[...] = dinv[:, None] * jnp.dot(
            x_ref[...], w1_ref[...], preferred_element_type=jnp.float32)

    return pl.pallas_call(
        body,
        grid=(npad // RB,),
        in_specs=[
            pl.BlockSpec((NC, RB), lambda i: (0, i)),
            pl.BlockSpec((RB, 128), lambda i: (i, 0)),
            pl.BlockSpec((128, 128), lambda i: (0, 0)),
        ],
        out_specs=pl.BlockSpec((RB, 128), lambda i: (i, 0)),
        out_shape=jax.ShapeDtypeStruct((npad, 128), jnp.float32),
    )(degp, xp, W1)


def _tc2(degp, s1, y1, b1, W2):
    npad = y1.shape[0]
    cd = W2.shape[1]

    def body(deg_ref, s1_ref, y1_ref, b1_ref, w2_ref, y2_ref):
        dinv = lax.rsqrt(deg_ref[0] + deg_ref[1] + 1.0)
        t = s1_ref[0] + s1_ref[1] + y1_ref[...]
        h = jnp.maximum(dinv[:, None] * t + b1_ref[...], 0.0)
        y2_ref[...] = dinv[:, None] * jnp.dot(
            h, w2_ref[...], preferred_element_type=jnp.float32)

    return pl.pallas_call(
        body,
        grid=(npad // RB,),
        in_specs=[
            pl.BlockSpec((NC, RB), lambda i: (0, i)),
            pl.BlockSpec((NC, RB, 128), lambda i: (0, i, 0)),
            pl.BlockSpec((RB, 128), lambda i: (i, 0)),
            pl.BlockSpec((128,), lambda i: (0,)),
            pl.BlockSpec((128, cd), lambda i: (0, 0)),
        ],
        out_specs=pl.BlockSpec((RB, cd), lambda i: (i, 0)),
        out_shape=jax.ShapeDtypeStruct((npad, cd), jnp.float32),
    )(degp, s1, y1, b1, W2)


def _tc3(degp, s2, y2, b2):
    npad = y2.shape[0]
    cd = y2.shape[1]

    def body(deg_ref, s2_ref, y2_ref, b2_ref, o_ref):
        dinv = lax.rsqrt(deg_ref[0] + deg_ref[1] + 1.0)
        o = dinv[:, None] * (s2_ref[0] + s2_ref[1] + y2_ref[...]) + b2_ref[...]
        m = jnp.max(o, axis=-1, keepdims=True)
        lse = jnp.log(jnp.sum(jnp.exp(o - m), axis=-1, keepdims=True)) + m
        o_ref[...] = o - lse

    return pl.pallas_call(
        body,
        grid=(npad // RB,),
        in_specs=[
            pl.BlockSpec((NC, RB), lambda i: (0, i)),
            pl.BlockSpec((NC, RB, cd), lambda i: (0, i, 0)),
            pl.BlockSpec((RB, cd), lambda i: (i, 0)),
            pl.BlockSpec((cd,), lambda i: (0,)),
        ],
        out_specs=pl.BlockSpec((RB, cd), lambda i: (i, 0)),
        out_shape=jax.ShapeDtypeStruct((npad, cd), jnp.float32),
    )(degp, s2, y2, b2)


def kernel(x, edge_index, W1, b1, W2, b2):
    n, d = x.shape
    e = edge_index.shape[1]

    npad = ((n + RB) // RB) * RB
    step = TILES * EB
    epad = ((e + step - 1) // step) * step
    nb = epad // step

    ei = edge_index.astype(jnp.int32)
    pad = jnp.full((epad - e,), n, jnp.int32)
    src_p = jnp.concatenate([ei[0], pad]).reshape(TILES, nb, EB)
    dst_p = jnp.concatenate([ei[1], pad]).reshape(TILES, nb, EB)
    xp = jnp.pad(x, ((0, npad - n), (0, 0)))

    degp = _make_deg(npad, nb)(dst_p)
    y1 = _tc1(degp, xp, W1)
    s1 = _make_prop(npad, nb, 128)(y1, src_p, dst_p)
    y2 = _tc2(degp, s1, y1, b1, W2)
    s2 = _make_prop(npad, nb, 16)(y2, src_p, dst_p)
    out = _tc3(degp, s2, y2, b2)
    return out[:n]
